# baseline (speedup 1.0000x reference)
import functools

import jax
import jax.numpy as jnp
from jax import lax
from jax.experimental import pallas as pl
from jax.experimental.pallas import tpu as pltpu
from jax.experimental.pallas import tpu_sc as plsc

NUM_EMB = 1000000
D = 64
BATCH = 4096
SEQ = 200
B_TOTAL = BATCH * SEQ
NC = 2
NS = 16
NW = NC * NS
BBLK = BATCH // NW
SP = 4
NSTEP = SEQ // SP
NBUF = 2
NGROUP = NSTEP // NBUF


def _embed_body(idxt_hbm, table_hbm, out_hbm, idx_v, rows_v, gsems, ssems):
    wid = lax.axis_index("s") * NC + lax.axis_index("c")
    b0 = wid * BBLK
    pltpu.sync_copy(idxt_hbm.at[:, pl.ds(b0, BBLK)], idx_v)

    def fire_gather(step, b):
        for j in range(SP):
            pltpu.async_copy(
                table_hbm.at[idx_v.at[step * SP + j, pl.ds(0, BBLK)]],
                rows_v.at[b, pl.ds(j * BBLK, BBLK)],
                gsems[b],
            )

    def wait_gather(b):
        pltpu.make_async_copy(
            table_hbm.at[pl.ds(0, SP * BBLK)], rows_v.at[b], gsems[b]
        ).wait()

    def fire_store(step, b):
        for j in range(SP):
            pltpu.async_copy(
                rows_v.at[b, pl.ds(j * BBLK, BBLK)],
                out_hbm.at[pl.ds((step * SP + j) * BATCH + b0, BBLK)],
                ssems[b],
            )

    def wait_store(b):
        pltpu.make_async_copy(
            rows_v.at[b], out_hbm.at[pl.ds(0, SP * BBLK)], ssems[b]
        ).wait()

    for b in range(NBUF):
        fire_gather(b, b)

    def group(g, carry):
        for b in range(NBUF):
            step = g * NBUF + b
            wait_gather(b)
            fire_store(step, b)
            wait_store(b)
            fire_gather(step + NBUF, b)
        return carry

    lax.fori_loop(0, NGROUP - 1, group, 0)

    for b in range(NBUF):
        step = (NGROUP - 1) * NBUF + b
        wait_gather(b)
        fire_store(step, b)
    for b in range(NBUF):
        wait_store(b)


@jax.jit
def _embed(idxt, embedding):
    mesh = plsc.VectorSubcoreMesh(
        core_axis_name="c", subcore_axis_name="s", num_cores=NC, num_subcores=NS
    )
    return pl.kernel(
        _embed_body,
        out_type=jax.ShapeDtypeStruct((B_TOTAL, D), jnp.float32),
        mesh=mesh,
        scratch_types=[
            pltpu.VMEM((SEQ, BBLK), jnp.int32),
            pltpu.VMEM((NBUF, SP * BBLK, D), jnp.float32),
            [pltpu.SemaphoreType.DMA] * NBUF,
            [pltpu.SemaphoreType.DMA] * NBUF,
        ],
        compiler_params=pltpu.CompilerParams(use_tc_tiling_on_sc=False),
    )(idxt, embedding)


def kernel(inputs, embedding):
    idxt = jnp.transpose(inputs)
    out = _embed(idxt, embedding)
    return out.reshape(SEQ, BATCH, D).transpose(1, 0, 2)


# speedup vs baseline: 1.3375x; 1.3375x over previous
import functools

import jax
import jax.numpy as jnp
from jax import lax
from jax.experimental import pallas as pl
from jax.experimental.pallas import tpu as pltpu
from jax.experimental.pallas import tpu_sc as plsc

NUM_EMB = 1000000
D = 64
BATCH = 4096
SEQ = 200
B_TOTAL = BATCH * SEQ
NC = 2
NS = 16
NW = NC * NS
BBLK = BATCH // NW
SP = 4
NSTEP = SEQ // SP
NBUF = 2
NGROUP = NSTEP // NBUF


def _embed_body(idxt_hbm, table_hbm, out_hbm, idx_v, rows_v, gsems, ssems):
    wid = lax.axis_index("s") * NC + lax.axis_index("c")
    b0 = wid * BBLK
    pltpu.sync_copy(idxt_hbm.at[:, pl.ds(b0, BBLK)], idx_v)

    def fire_gather(step, b):
        for j in range(SP):
            pltpu.async_copy(
                table_hbm.at[idx_v.at[step * SP + j, pl.ds(0, BBLK)]],
                rows_v.at[b, pl.ds(j * BBLK, BBLK)],
                gsems[b],
            )

    def wait_gather(b):
        pltpu.make_async_copy(
            table_hbm.at[pl.ds(0, SP * BBLK)], rows_v.at[b], gsems[b]
        ).wait()

    def fire_store(step, b):
        for j in range(SP):
            pltpu.async_copy(
                rows_v.at[b, pl.ds(j * BBLK, BBLK)],
                out_hbm.at[pl.ds((step * SP + j) * BATCH + b0, BBLK), pl.ds(0, D)],
                ssems[b],
            )

    def wait_store(b):
        pltpu.make_async_copy(
            rows_v.at[b], out_hbm.at[pl.ds(0, SP * BBLK), pl.ds(0, D)], ssems[b]
        ).wait()

    for b in range(NBUF):
        fire_gather(b, b)

    def group(g, carry):
        for b in range(NBUF):
            step = g * NBUF + b
            wait_gather(b)
            fire_store(step, b)
            wait_store(b)
            fire_gather(step + NBUF, b)
        return carry

    lax.fori_loop(0, NGROUP - 1, group, 0)

    for b in range(NBUF):
        step = (NGROUP - 1) * NBUF + b
        wait_gather(b)
        fire_store(step, b)
    for b in range(NBUF):
        wait_store(b)


@jax.jit
def _embed(idxt, embedding):
    mesh = plsc.VectorSubcoreMesh(
        core_axis_name="c", subcore_axis_name="s", num_cores=NC, num_subcores=NS
    )
    return pl.kernel(
        _embed_body,
        out_type=jax.ShapeDtypeStruct((B_TOTAL, 128), jnp.float32),
        mesh=mesh,
        scratch_types=[
            pltpu.VMEM((SEQ, BBLK), jnp.int32),
            pltpu.VMEM((NBUF, SP * BBLK, D), jnp.float32),
            [pltpu.SemaphoreType.DMA] * NBUF,
            [pltpu.SemaphoreType.DMA] * NBUF,
        ],
        compiler_params=pltpu.CompilerParams(use_tc_tiling_on_sc=False),
    )(idxt, embedding)


def kernel(inputs, embedding):
    idxt = jnp.transpose(inputs)
    out = _embed(idxt, embedding)
    return out[:, :D].reshape(SEQ, BATCH, D).transpose(1, 0, 2)
